# Initial kernel scaffold; baseline (speedup 1.0000x reference)
import jax
import jax.numpy as jnp
from jax.experimental import pallas as pl


def kernel(nodes, edges, globals_, params, senders, receivers, n_node, n_edge, gumbel_temperature):
    raise NotImplementedError("write your pallas kernel here")


# speedup vs baseline: 4.8215x; 4.8215x over previous
import functools

import jax
import jax.numpy as jnp
from jax import lax
from jax.experimental import pallas as pl
from jax.experimental.pallas import tpu as pltpu
from jax.experimental.pallas import tpu_sc as plsc

_N = 10000
_E = 320000
_DF = 128
_DE = 16
_DG = 128
_LAT = 64
_MEI = 32

_F32 = jnp.float32

_NW = 32
_GPW = (2 * _E) // _NW
_GCH = 80
_GNCH = _GPW // _GCH


def _sc_gather(idx, table):
    mesh = plsc.VectorSubcoreMesh(core_axis_name="c", subcore_axis_name="s")

    @functools.partial(
        pl.kernel,
        out_type=jax.ShapeDtypeStruct((2 * _E, _DF), _F32),
        mesh=mesh,
        scratch_types=[
            pltpu.VMEM((_GPW,), jnp.int32),
            pltpu.VMEM((_GCH, _DF), _F32),
            pltpu.VMEM((_GCH, _DF), _F32),
            pltpu.SemaphoreType.DMA,
            pltpu.SemaphoreType.DMA,
        ],
    )
    def k(idx_hbm, tbl_hbm, out_hbm, idx_v, buf0, buf1, sem0, sem1):
        wid = lax.axis_index("s") * 2 + lax.axis_index("c")
        base = wid * _GPW
        pltpu.sync_copy(idx_hbm.at[pl.ds(base, _GPW)], idx_v)

        def start(j, buf, sem):
            pltpu.async_copy(tbl_hbm.at[idx_v.at[pl.ds(j * _GCH, _GCH)]], buf, sem)

        def wait(j, buf, sem):
            pltpu.make_async_copy(
                tbl_hbm.at[idx_v.at[pl.ds(j * _GCH, _GCH)]], buf, sem).wait()

        start(0, buf0, sem0)

        def body(it, _):
            j0 = it * 2
            start(j0 + 1, buf1, sem1)
            wait(j0, buf0, sem0)
            pltpu.sync_copy(buf0, out_hbm.at[pl.ds(base + j0 * _GCH, _GCH)])

            @pl.when(it + 1 < _GNCH // 2)
            def _():
                start(j0 + 2, buf0, sem0)

            wait(j0 + 1, buf1, sem1)
            pltpu.sync_copy(buf1, out_hbm.at[pl.ds(base + (j0 + 1) * _GCH, _GCH)])
            return 0

        lax.fori_loop(0, _GNCH // 2, body, 0)

    return k(idx, table)


_SPW = _E // _NW
_SCH = 80
_SNCH = _SPW // _SCH
_RPT = 624
_RTAIL = _N - 16 * _RPT


def _sc_scatter(e2, recv, zeros):
    mesh = plsc.VectorSubcoreMesh(core_axis_name="c", subcore_axis_name="s")

    @functools.partial(
        pl.kernel,
        out_type=jax.ShapeDtypeStruct((2, _N, _DF), _F32),
        mesh=mesh,
        scratch_types=[
            pltpu.VMEM_SHARED((_N, _DF), _F32),
            pltpu.VMEM((_SCH,), jnp.int32),
            pltpu.VMEM((_SCH, _DF), _F32),
        ],
    )
    def k(e2_hbm, recv_hbm, zeros_hbm, out_hbm, shared, idx_v, rows_v):
        cid = lax.axis_index("c")
        sid = lax.axis_index("s")
        pltpu.sync_copy(zeros_hbm.at[pl.ds(sid * _RPT, _RPT)],
                        shared.at[pl.ds(sid * _RPT, _RPT)])

        @pl.when(sid == 0)
        def _():
            pltpu.sync_copy(zeros_hbm.at[pl.ds(16 * _RPT, _RTAIL)],
                            shared.at[pl.ds(16 * _RPT, _RTAIL)])

        plsc.subcore_barrier()
        base = (sid * 2 + cid) * _SPW

        def body(j, _):
            off = base + j * _SCH
            pltpu.sync_copy(recv_hbm.at[pl.ds(off, _SCH)], idx_v)
            pltpu.sync_copy(e2_hbm.at[pl.ds(off, _SCH)], rows_v)
            pltpu.sync_copy(rows_v, shared.at[idx_v], add=True)
            return 0

        lax.fori_loop(0, _SNCH, body, 0)
        plsc.subcore_barrier()
        pltpu.sync_copy(shared.at[pl.ds(sid * _RPT, _RPT)],
                        out_hbm.at[cid].at[pl.ds(sid * _RPT, _RPT)])

        @pl.when(sid == 0)
        def _():
            pltpu.sync_copy(shared.at[pl.ds(16 * _RPT, _RTAIL)],
                            out_hbm.at[cid].at[pl.ds(16 * _RPT, _RTAIL)])

    return k(e2, recv, zeros)


_BE = 3200
_GSTEPS = _E // _BE


def _tc_edge(gn, edges, globals_, Ws, Wr, We, Wg, b1, W2blk, b2):
    def body(gs_ref, gr_ref, ed_ref, gl_ref, Ws_ref, Wr_ref, We_ref, Wg_ref,
             b1_ref, W2_ref, b2_ref, e2_ref, sum_ref):
        c = jnp.dot(gl_ref[...], Wg_ref[...], preferred_element_type=_F32) + b1_ref[...]
        pre = (jnp.dot(gs_ref[...], Ws_ref[...], preferred_element_type=_F32)
               + jnp.dot(gr_ref[...], Wr_ref[...], preferred_element_type=_F32)
               + jnp.dot(ed_ref[...], We_ref[...], preferred_element_type=_F32)
               + c)
        e2 = jnp.dot(jnp.maximum(pre, 0.0), W2_ref[...],
                     preferred_element_type=_F32) + b2_ref[...]
        e2_ref[...] = e2
        s = jnp.sum(e2, axis=0, keepdims=True)

        @pl.when(pl.program_id(0) == 0)
        def _():
            sum_ref[...] = s

        @pl.when(pl.program_id(0) > 0)
        def _():
            sum_ref[...] = sum_ref[...] + s

    full = lambda a: pl.BlockSpec(a.shape, lambda i: (0,) * a.ndim)
    return pl.pallas_call(
        body,
        grid=(_GSTEPS,),
        in_specs=[
            pl.BlockSpec((_BE, _DF), lambda i: (i, 0)),
            pl.BlockSpec((_BE, _DF), lambda i: (i + _GSTEPS, 0)),
            pl.BlockSpec((_BE, _DE), lambda i: (i, 0)),
            full(globals_), full(Ws), full(Wr), full(We), full(Wg),
            full(b1), full(W2blk), full(b2),
        ],
        out_specs=[
            pl.BlockSpec((_BE, 2 * _LAT), lambda i: (i, 0)),
            pl.BlockSpec((1, 2 * _LAT), lambda i: (0, 0)),
        ],
        out_shape=[
            jax.ShapeDtypeStruct((_E, 2 * _LAT), _F32),
            jax.ShapeDtypeStruct((1, 2 * _LAT), _F32),
        ],
    )(gn, gn, edges, globals_, Ws, Wr, We, Wg, b1, W2blk, b2)


def _tc_dense1(nodes, ag0, ag1, e2sum, globals_, eps, nnne,
               Wnn, Wna, Wng, b1n, W2n, b2n,
               G1e, b1ge, W2ge, b2ge, G1s, b1gs, W2gs, b2gs,
               Wz, wnn_row, wne_row, wpos_row, b1d, W2d, b2d,
               W1ei_a, W1ei_b, b1ei, Wme_s, Wme_d, b1me):
    def body(nodes_ref, ag0_ref, ag1_ref, e2sum_ref, gl_ref, eps_ref, nnne_ref,
             Wnn_ref, Wna_ref, Wng_ref, b1n_ref, W2n_ref, b2n_ref,
             G1e_ref, b1ge_ref, W2ge_ref, b2ge_ref,
             G1s_ref, b1gs_ref, W2gs_ref, b2gs_ref,
             Wz_ref, wnn_ref, wne_ref, wpos_ref, b1d_ref, W2d_ref, b2d_ref,
             Wa_ref, Wb_ref, b1ei_ref, Wms_ref, Wmd_ref, b1me_ref,
             mu_ref, ls_ref, nf_ref, A_ref, B_ref, C_ref, D_ref):
        gl = gl_ref[...]
        agg = ag0_ref[...] + ag1_ref[...]
        npre = (jnp.dot(nodes_ref[...], Wnn_ref[...], preferred_element_type=_F32)
                + jnp.dot(agg, Wna_ref[...], preferred_element_type=_F32)
                + jnp.dot(gl, Wng_ref[...], preferred_element_type=_F32)
                + b1n_ref[...])
        n2 = jnp.dot(jnp.maximum(npre, 0.0), W2n_ref[...],
                     preferred_element_type=_F32) + b2n_ref[...]
        n2sum = jnp.sum(n2, axis=0, keepdims=True)
        e2sum = e2sum_ref[...]

        def glob(ns, es, G1, b1g, W2g, b2g):
            gin = jnp.concatenate([ns, es, gl], axis=1)
            gpre = jnp.dot(gin, G1, preferred_element_type=_F32) + b1g
            return jnp.dot(jnp.maximum(gpre, 0.0), W2g,
                           preferred_element_type=_F32) + b2g

        mu = glob(n2sum[:, :_LAT], e2sum[:, :_LAT],
                  G1e_ref[...], b1ge_ref[...], W2ge_ref[...], b2ge_ref[...])
        ls = glob(n2sum[:, _LAT:], e2sum[:, _LAT:],
                  G1s_ref[...], b1gs_ref[...], W2gs_ref[...], b2gs_ref[...])
        mu_ref[...] = mu
        ls_ref[...] = ls

        z = mu + jnp.exp(ls) * eps_ref[...]
        nn = nnne_ref[0, 0]
        ne = nnne_ref[0, 1]
        zc = (jnp.dot(z, Wz_ref[...], preferred_element_type=_F32)
              + nn * wnn_ref[...] + ne * wne_ref[...] + b1d_ref[...])
        pos = lax.broadcasted_iota(jnp.int32, (_N, 1), 0).astype(_F32) * (1.0 / _N)
        nf = jnp.dot(jnp.maximum(zc + pos * wpos_ref[...], 0.0), W2d_ref[...],
                     preferred_element_type=_F32) + b2d_ref[...]
        nf_ref[...] = nf
        A_ref[...] = jnp.dot(nf, Wa_ref[...], preferred_element_type=_F32) + b1ei_ref[...]
        B_ref[...] = jnp.dot(nf, Wb_ref[...], preferred_element_type=_F32)
        C_ref[...] = jnp.dot(nf, Wms_ref[...], preferred_element_type=_F32) + b1me_ref[...]
        D_ref[...] = jnp.dot(nf, Wmd_ref[...], preferred_element_type=_F32)

    return pl.pallas_call(
        body,
        out_shape=[
            jax.ShapeDtypeStruct((1, _LAT), _F32),
            jax.ShapeDtypeStruct((1, _LAT), _F32),
            jax.ShapeDtypeStruct((_N, _DF), _F32),
            jax.ShapeDtypeStruct((_N, _DF), _F32),
            jax.ShapeDtypeStruct((_N, _DF), _F32),
            jax.ShapeDtypeStruct((_N, _DF), _F32),
            jax.ShapeDtypeStruct((_N, _DF), _F32),
        ],
    )(nodes, ag0, ag1, e2sum, globals_, eps, nnne,
      Wnn, Wna, Wng, b1n, W2n, b2n,
      G1e, b1ge, W2ge, b2ge, G1s, b1gs, W2gs, b2gs,
      Wz, wnn_row, wne_row, wpos_row, b1d, W2d, b2d,
      W1ei_a, W1ei_b, b1ei, Wme_s, Wme_d, b1me)


def _tc_dense2(A, B2, C, D2, u0, u1,
               W2ei, b2ei, Wmef, W2me, b2me, wld, bld, temp):
    def body(A_ref, B2_ref, C_ref, D2_ref, u0_ref, u1_ref,
             W2ei_ref, b2ei_ref, Wmef_ref, W2me_ref, b2me_ref,
             wld_ref, bld_ref, tmp_ref,
             ew_ref, aggd_ref, agg2):
        kidx = pl.program_id(0)
        k = kidx + 1

        @pl.when(kidx == 0)
        def _():
            agg2[...] = jnp.zeros((_N + _MEI, _LAT), _F32)
            ew_ref[...] = jnp.zeros((_N, _MEI), _F32)

        Bk = B2_ref[pl.ds(k, _N), :]
        Dk = D2_ref[pl.ds(k, _N), :]
        ef = jnp.dot(jnp.maximum(A_ref[...] + Bk, 0.0), W2ei_ref[...],
                     preferred_element_type=_F32) + b2ei_ref[...]
        mpre = jnp.dot(ef, Wmef_ref[...], preferred_element_type=_F32) + C_ref[...] + Dk
        m = jnp.dot(jnp.maximum(mpre, 0.0), W2me_ref[...],
                    preferred_element_type=_F32) + b2me_ref[...]
        agg2[pl.ds(k, _N), :] = agg2[pl.ds(k, _N), :] + m

        ohrow = (lax.broadcasted_iota(jnp.int32, (1, _MEI), 1) == kidx).astype(_F32)
        u0k = jnp.sum(u0_ref[...] * ohrow, axis=1, keepdims=True)
        u1k = jnp.sum(u1_ref[...] * ohrow, axis=1, keepdims=True)
        g0 = -jnp.log(-jnp.log(u0k))
        g1 = -jnp.log(-jnp.log(u1k))
        d = (jnp.sum(m * wld_ref[...], axis=1, keepdims=True)
             + bld_ref[0, 0] + g1 - g0) / tmp_ref[0, 0]
        sig = 1.0 / (1.0 + jnp.exp(-d))
        ew_ref[...] = ew_ref[...] + sig * ohrow

        @pl.when(kidx == _MEI - 1)
        def _():
            tail = jnp.concatenate(
                [agg2[pl.ds(_N, _MEI), :],
                 jnp.zeros((_N - _MEI, _LAT), _F32)], axis=0)
            aggd_ref[...] = agg2[pl.ds(0, _N), :] + tail

    full = lambda a: pl.BlockSpec(a.shape, lambda i: (0,) * a.ndim)
    args = (A, B2, C, D2, u0, u1,
            W2ei, b2ei, Wmef, W2me, b2me, wld, bld, temp)
    return pl.pallas_call(
        body,
        grid=(_MEI,),
        in_specs=[full(a) for a in args],
        out_specs=[
            pl.BlockSpec((_N, _MEI), lambda i: (0, 0)),
            pl.BlockSpec((_N, _LAT), lambda i: (0, 0)),
        ],
        out_shape=[
            jax.ShapeDtypeStruct((_N, _MEI), _F32),
            jax.ShapeDtypeStruct((_N, _LAT), _F32),
        ],
        scratch_shapes=[pltpu.VMEM((_N + _MEI, _LAT), _F32)],
    )(*args)


def _tc_dense3(nfeat, aggd, W1mnf, W1mna, b1mn, W2mn, b2mn, Wrc, brc):
    def body(nf_ref, aggd_ref, W1mnf_ref, W1mna_ref, b1mn_ref,
             W2mn_ref, b2mn_ref, Wrc_ref, brc_ref, rc_ref):
        n2pre = (jnp.dot(nf_ref[...], W1mnf_ref[...], preferred_element_type=_F32)
                 + jnp.dot(aggd_ref[...], W1mna_ref[...], preferred_element_type=_F32)
                 + b1mn_ref[...])
        n2d = jnp.dot(jnp.maximum(n2pre, 0.0), W2mn_ref[...],
                      preferred_element_type=_F32) + b2mn_ref[...]
        rc_ref[...] = jnp.dot(n2d, Wrc_ref[...],
                              preferred_element_type=_F32) + brc_ref[...]

    return pl.pallas_call(
        body,
        out_shape=jax.ShapeDtypeStruct((_N, _DF), _F32),
    )(nfeat, aggd, W1mnf, W1mna, b1mn, W2mn, b2mn, Wrc, brc)


def kernel(nodes, edges, globals_, params, senders, receivers,
           n_node, n_edge, gumbel_temperature):
    nodes = nodes.astype(_F32)
    edges = edges.astype(_F32)
    globals_ = globals_.astype(_F32)
    s32 = senders.astype(jnp.int32)
    r32 = receivers.astype(jnp.int32)

    def edge_parts(p):
        W1, b1 = p['edge'][0]
        return (W1[:_DE], W1[_DE:_DE + _DF], W1[_DE + _DF:_DE + 2 * _DF],
                W1[_DE + 2 * _DF:], b1)
    We_e, Ws_e, Wr_e, Wg_e, b1_e = edge_parts(params['enc'])
    We_s, Ws_s, Wr_s, Wg_s, b1_s = edge_parts(params['sig'])
    We = jnp.concatenate([We_e, We_s], 1)
    Ws = jnp.concatenate([Ws_e, Ws_s], 1)
    Wr = jnp.concatenate([Wr_e, Wr_s], 1)
    Wg = jnp.concatenate([Wg_e, Wg_s], 1)
    b1 = jnp.concatenate([b1_e, b1_s]).reshape(1, -1)
    W2e_e, b2e_e = params['enc']['edge'][1]
    W2e_s, b2e_s = params['sig']['edge'][1]
    W2blk = jnp.zeros((2 * _DF, 2 * _LAT), _F32)
    W2blk = W2blk.at[:_DF, :_LAT].set(W2e_e).at[_DF:, _LAT:].set(W2e_s)
    b2 = jnp.concatenate([b2e_e, b2e_s]).reshape(1, -1)

    idx = jnp.concatenate([s32, r32], 0)
    gn = _sc_gather(idx, nodes)
    e2, e2sum = _tc_edge(gn, edges, globals_, Ws, Wr, We, Wg, b1, W2blk, b2)
    parts = _sc_scatter(e2, r32, jnp.zeros((_N, _DF), _F32))

    W1n_e, b1n_e = params['enc']['node'][0]
    W1n_s, b1n_s = params['sig']['node'][0]
    W2n_e, b2n_e = params['enc']['node'][1]
    W2n_s, b2n_s = params['sig']['node'][1]
    Wnn = jnp.concatenate([W1n_e[:_DF], W1n_s[:_DF]], 1)
    Wna = jnp.zeros((2 * _LAT, 2 * _DF), _F32)
    Wna = Wna.at[:_LAT, :_DF].set(W1n_e[_DF:_DF + _LAT])
    Wna = Wna.at[_LAT:, _DF:].set(W1n_s[_DF:_DF + _LAT])
    Wng = jnp.concatenate([W1n_e[_DF + _LAT:], W1n_s[_DF + _LAT:]], 1)
    b1n = jnp.concatenate([b1n_e, b1n_s]).reshape(1, -1)
    W2n = jnp.zeros((2 * _DF, 2 * _LAT), _F32)
    W2n = W2n.at[:_DF, :_LAT].set(W2n_e).at[_DF:, _LAT:].set(W2n_s)
    b2n = jnp.concatenate([b2n_e, b2n_s]).reshape(1, -1)

    G1e, b1ge = params['enc']['glob'][0]
    W2ge, b2ge = params['enc']['glob'][1]
    G1s, b1gs = params['sig']['glob'][0]
    W2gs, b2gs = params['sig']['glob'][1]

    dp = params['dec']
    W1d, b1d = dp['node_init'][0]
    W2d, b2d = dp['node_init'][1]
    Wz = W1d[:_LAT]
    wnn_row = W1d[_LAT].reshape(1, -1)
    wne_row = W1d[_LAT + 1].reshape(1, -1)
    wpos_row = W1d[_LAT + 2].reshape(1, -1)
    W1ei, b1ei = dp['edge_init'][0]
    W2ei, b2ei = dp['edge_init'][1]
    W1me, b1me = dp['mpg_edge'][0]
    W2me, b2me = dp['mpg_edge'][1]

    eps = jax.random.normal(jax.random.key(42), (1, _LAT), dtype=_F32)
    nnne = jnp.zeros((1, _DF), _F32)
    nnne = nnne.at[0, 0].set(n_node.astype(_F32)[0])
    nnne = nnne.at[0, 1].set(n_edge.astype(_F32)[0])

    mu, ls, nfeat, A, B, C, D = _tc_dense1(
        nodes, parts[0], parts[1], e2sum, globals_, eps, nnne,
        Wnn, Wna, Wng, b1n, W2n, b2n,
        G1e, b1ge.reshape(1, -1), W2ge, b2ge.reshape(1, -1),
        G1s, b1gs.reshape(1, -1), W2gs, b2gs.reshape(1, -1),
        Wz, wnn_row, wne_row, wpos_row, b1d.reshape(1, -1), W2d, b2d.reshape(1, -1),
        W1ei[:_DF], W1ei[_DF:], b1ei.reshape(1, -1),
        W1me[_DE:_DE + _DF], W1me[_DE + _DF:], b1me.reshape(1, -1))

    B2 = jnp.concatenate([B, B[:_MEI]], 0)
    D2 = jnp.concatenate([D, D[:_MEI]], 0)

    Wl, bl = dp['logit'][0]
    wld = (Wl[:, 1] - Wl[:, 0]).reshape(1, -1)
    bld = (bl[1] - bl[0]).reshape(1, 1)
    W1mn, b1mn = dp['mpg_node'][0]
    W2mn, b2mn = dp['mpg_node'][1]
    Wrc, brc = dp['recon'][0]

    u = jax.random.uniform(jax.random.key(43), (_N * _MEI, 2),
                           minval=1e-6, maxval=1.0 - 1e-6)
    u3 = u.reshape(_MEI, _N, 2)
    u0 = u3[:, :, 0].T
    u1 = u3[:, :, 1].T
    temp = jnp.asarray(gumbel_temperature, _F32).reshape(1, 1)

    ew, aggd = _tc_dense2(
        A, B2, C, D2, u0, u1,
        W2ei, b2ei.reshape(1, -1), W1me[:_DE], W2me, b2me.reshape(1, -1),
        wld, bld, temp)
    recon = _tc_dense3(
        nfeat, aggd, W1mn[:_DF], W1mn[_DF:], b1mn.reshape(1, -1),
        W2mn, b2mn.reshape(1, -1), Wrc, brc.reshape(1, -1))

    edge_weights = ew.T.reshape(-1)
    return (recon, edge_weights, mu, ls)


# speedup vs baseline: 4.9835x; 1.0336x over previous
import functools

import jax
import jax.numpy as jnp
from jax import lax
from jax.experimental import pallas as pl
from jax.experimental.pallas import tpu as pltpu
from jax.experimental.pallas import tpu_sc as plsc

_N = 10000
_E = 320000
_DF = 128
_DE = 16
_DG = 128
_LAT = 64
_MEI = 32

_F32 = jnp.float32

_NW = 32
_GPW = (2 * _E) // _NW
_GCH = 80
_GNCH = _GPW // _GCH


def _sc_gather(idx, table):
    mesh = plsc.VectorSubcoreMesh(core_axis_name="c", subcore_axis_name="s")

    @functools.partial(
        pl.kernel,
        out_type=jax.ShapeDtypeStruct((2 * _E, _DF), _F32),
        mesh=mesh,
        scratch_types=[
            pltpu.VMEM((_GPW,), jnp.int32),
            pltpu.VMEM((_GCH, _DF), _F32),
            pltpu.VMEM((_GCH, _DF), _F32),
            pltpu.SemaphoreType.DMA,
            pltpu.SemaphoreType.DMA,
        ],
    )
    def k(idx_hbm, tbl_hbm, out_hbm, idx_v, buf0, buf1, sem0, sem1):
        wid = lax.axis_index("s") * 2 + lax.axis_index("c")
        base = wid * _GPW
        pltpu.sync_copy(idx_hbm.at[pl.ds(base, _GPW)], idx_v)

        def start(j, buf, sem):
            pltpu.async_copy(tbl_hbm.at[idx_v.at[pl.ds(j * _GCH, _GCH)]], buf, sem)

        def wait(j, buf, sem):
            pltpu.make_async_copy(
                tbl_hbm.at[idx_v.at[pl.ds(j * _GCH, _GCH)]], buf, sem).wait()

        start(0, buf0, sem0)

        def body(it, _):
            j0 = it * 2
            start(j0 + 1, buf1, sem1)
            wait(j0, buf0, sem0)
            pltpu.sync_copy(buf0, out_hbm.at[pl.ds(base + j0 * _GCH, _GCH)])

            @pl.when(it + 1 < _GNCH // 2)
            def _():
                start(j0 + 2, buf0, sem0)

            wait(j0 + 1, buf1, sem1)
            pltpu.sync_copy(buf1, out_hbm.at[pl.ds(base + (j0 + 1) * _GCH, _GCH)])
            return 0

        lax.fori_loop(0, _GNCH // 2, body, 0)

    return k(idx, table)


_SPW = _E // _NW
_SCH = 80
_SNCH = _SPW // _SCH
_RPT = 624
_RTAIL = _N - 16 * _RPT


def _sc_scatter(e2, recv, zeros):
    mesh = plsc.VectorSubcoreMesh(core_axis_name="c", subcore_axis_name="s")

    @functools.partial(
        pl.kernel,
        out_type=jax.ShapeDtypeStruct((2, _N, _DF), _F32),
        mesh=mesh,
        scratch_types=[
            pltpu.VMEM_SHARED((_N, _DF), _F32),
            pltpu.VMEM((_SCH,), jnp.int32),
            pltpu.VMEM((_SCH, _DF), _F32),
        ],
    )
    def k(e2_hbm, recv_hbm, zeros_hbm, out_hbm, shared, idx_v, rows_v):
        cid = lax.axis_index("c")
        sid = lax.axis_index("s")
        pltpu.sync_copy(zeros_hbm.at[pl.ds(sid * _RPT, _RPT)],
                        shared.at[pl.ds(sid * _RPT, _RPT)])

        @pl.when(sid == 0)
        def _():
            pltpu.sync_copy(zeros_hbm.at[pl.ds(16 * _RPT, _RTAIL)],
                            shared.at[pl.ds(16 * _RPT, _RTAIL)])

        plsc.subcore_barrier()
        base = (sid * 2 + cid) * _SPW

        def body(j, _):
            off = base + j * _SCH
            pltpu.sync_copy(recv_hbm.at[pl.ds(off, _SCH)], idx_v)
            pltpu.sync_copy(e2_hbm.at[pl.ds(off, _SCH)], rows_v)
            pltpu.sync_copy(rows_v, shared.at[idx_v], add=True)
            return 0

        lax.fori_loop(0, _SNCH, body, 0)
        plsc.subcore_barrier()
        pltpu.sync_copy(shared.at[pl.ds(sid * _RPT, _RPT)],
                        out_hbm.at[cid].at[pl.ds(sid * _RPT, _RPT)])

        @pl.when(sid == 0)
        def _():
            pltpu.sync_copy(shared.at[pl.ds(16 * _RPT, _RTAIL)],
                            out_hbm.at[cid].at[pl.ds(16 * _RPT, _RTAIL)])

    return k(e2, recv, zeros)


_BE = 3200
_GSTEPS = _E // _BE


def _tc_edge(gn, edges, globals_, Ws, Wr, We, Wg, b1, W2blk, b2):
    def body(gs_ref, gr_ref, ed_ref, gl_ref, Ws_ref, Wr_ref, We_ref, Wg_ref,
             b1_ref, W2_ref, b2_ref, e2_ref, sum_ref):
        bf = jnp.bfloat16
        c = jnp.dot(gl_ref[...], Wg_ref[...], preferred_element_type=_F32) + b1_ref[...]
        pre = (jnp.dot(gs_ref[...].astype(bf), Ws_ref[...].astype(bf),
                       preferred_element_type=_F32)
               + jnp.dot(gr_ref[...].astype(bf), Wr_ref[...].astype(bf),
                         preferred_element_type=_F32)
               + jnp.dot(ed_ref[...].astype(bf), We_ref[...].astype(bf),
                         preferred_element_type=_F32)
               + c)
        e2 = jnp.dot(jnp.maximum(pre, 0.0).astype(bf), W2_ref[...].astype(bf),
                     preferred_element_type=_F32) + b2_ref[...]
        e2_ref[...] = e2
        s = jnp.sum(e2, axis=0, keepdims=True)

        @pl.when(pl.program_id(0) == 0)
        def _():
            sum_ref[...] = s

        @pl.when(pl.program_id(0) > 0)
        def _():
            sum_ref[...] = sum_ref[...] + s

    full = lambda a: pl.BlockSpec(a.shape, lambda i: (0,) * a.ndim)
    return pl.pallas_call(
        body,
        grid=(_GSTEPS,),
        in_specs=[
            pl.BlockSpec((_BE, _DF), lambda i: (i, 0)),
            pl.BlockSpec((_BE, _DF), lambda i: (i + _GSTEPS, 0)),
            pl.BlockSpec((_BE, _DE), lambda i: (i, 0)),
            full(globals_), full(Ws), full(Wr), full(We), full(Wg),
            full(b1), full(W2blk), full(b2),
        ],
        out_specs=[
            pl.BlockSpec((_BE, 2 * _LAT), lambda i: (i, 0)),
            pl.BlockSpec((1, 2 * _LAT), lambda i: (0, 0)),
        ],
        out_shape=[
            jax.ShapeDtypeStruct((_E, 2 * _LAT), _F32),
            jax.ShapeDtypeStruct((1, 2 * _LAT), _F32),
        ],
    )(gn, gn, edges, globals_, Ws, Wr, We, Wg, b1, W2blk, b2)


def _tc_dense1(nodes, ag0, ag1, e2sum, globals_, eps, nnne,
               Wnn, Wna, Wng, b1n, W2n, b2n,
               G1e, b1ge, W2ge, b2ge, G1s, b1gs, W2gs, b2gs,
               Wz, wnn_row, wne_row, wpos_row, b1d, W2d, b2d,
               W1ei_a, W1ei_b, b1ei, Wme_s, Wme_d, b1me):
    def body(nodes_ref, ag0_ref, ag1_ref, e2sum_ref, gl_ref, eps_ref, nnne_ref,
             Wnn_ref, Wna_ref, Wng_ref, b1n_ref, W2n_ref, b2n_ref,
             G1e_ref, b1ge_ref, W2ge_ref, b2ge_ref,
             G1s_ref, b1gs_ref, W2gs_ref, b2gs_ref,
             Wz_ref, wnn_ref, wne_ref, wpos_ref, b1d_ref, W2d_ref, b2d_ref,
             Wa_ref, Wb_ref, b1ei_ref, Wms_ref, Wmd_ref, b1me_ref,
             mu_ref, ls_ref, nf_ref, A_ref, B_ref, C_ref, D_ref):
        gl = gl_ref[...]
        agg = ag0_ref[...] + ag1_ref[...]
        npre = (jnp.dot(nodes_ref[...], Wnn_ref[...], preferred_element_type=_F32)
                + jnp.dot(agg, Wna_ref[...], preferred_element_type=_F32)
                + jnp.dot(gl, Wng_ref[...], preferred_element_type=_F32)
                + b1n_ref[...])
        n2 = jnp.dot(jnp.maximum(npre, 0.0), W2n_ref[...],
                     preferred_element_type=_F32) + b2n_ref[...]
        n2sum = jnp.sum(n2, axis=0, keepdims=True)
        e2sum = e2sum_ref[...]

        def glob(ns, es, G1, b1g, W2g, b2g):
            gin = jnp.concatenate([ns, es, gl], axis=1)
            gpre = jnp.dot(gin, G1, preferred_element_type=_F32) + b1g
            return jnp.dot(jnp.maximum(gpre, 0.0), W2g,
                           preferred_element_type=_F32) + b2g

        mu = glob(n2sum[:, :_LAT], e2sum[:, :_LAT],
                  G1e_ref[...], b1ge_ref[...], W2ge_ref[...], b2ge_ref[...])
        ls = glob(n2sum[:, _LAT:], e2sum[:, _LAT:],
                  G1s_ref[...], b1gs_ref[...], W2gs_ref[...], b2gs_ref[...])
        mu_ref[...] = mu
        ls_ref[...] = ls

        z = mu + jnp.exp(ls) * eps_ref[...]
        nn = nnne_ref[0, 0]
        ne = nnne_ref[0, 1]
        zc = (jnp.dot(z, Wz_ref[...], preferred_element_type=_F32)
              + nn * wnn_ref[...] + ne * wne_ref[...] + b1d_ref[...])
        pos = lax.broadcasted_iota(jnp.int32, (_N, 1), 0).astype(_F32) * (1.0 / _N)
        nf = jnp.dot(jnp.maximum(zc + pos * wpos_ref[...], 0.0), W2d_ref[...],
                     preferred_element_type=_F32) + b2d_ref[...]
        nf_ref[...] = nf
        A_ref[...] = jnp.dot(nf, Wa_ref[...], preferred_element_type=_F32) + b1ei_ref[...]
        B_ref[...] = jnp.dot(nf, Wb_ref[...], preferred_element_type=_F32)
        C_ref[...] = jnp.dot(nf, Wms_ref[...], preferred_element_type=_F32) + b1me_ref[...]
        D_ref[...] = jnp.dot(nf, Wmd_ref[...], preferred_element_type=_F32)

    return pl.pallas_call(
        body,
        out_shape=[
            jax.ShapeDtypeStruct((1, _LAT), _F32),
            jax.ShapeDtypeStruct((1, _LAT), _F32),
            jax.ShapeDtypeStruct((_N, _DF), _F32),
            jax.ShapeDtypeStruct((_N, _DF), _F32),
            jax.ShapeDtypeStruct((_N, _DF), _F32),
            jax.ShapeDtypeStruct((_N, _DF), _F32),
            jax.ShapeDtypeStruct((_N, _DF), _F32),
        ],
    )(nodes, ag0, ag1, e2sum, globals_, eps, nnne,
      Wnn, Wna, Wng, b1n, W2n, b2n,
      G1e, b1ge, W2ge, b2ge, G1s, b1gs, W2gs, b2gs,
      Wz, wnn_row, wne_row, wpos_row, b1d, W2d, b2d,
      W1ei_a, W1ei_b, b1ei, Wme_s, Wme_d, b1me)


def _tc_dense2(A, B2, C, D2, u0, u1,
               W2ei, b2ei, Wmef, W2me, b2me, wld, bld, temp):
    def body(A_ref, B2_ref, C_ref, D2_ref, u0_ref, u1_ref,
             W2ei_ref, b2ei_ref, Wmef_ref, W2me_ref, b2me_ref,
             wld_ref, bld_ref, tmp_ref,
             ew_ref, aggd_ref, agg2, gd_scr):
        kidx = pl.program_id(0)
        k = kidx + 1
        bf = jnp.bfloat16

        @pl.when(kidx == 0)
        def _():
            agg2[...] = jnp.zeros((_N + _MEI, _LAT), _F32)
            ew_ref[...] = jnp.zeros((_N, _MEI), _F32)
            gd_scr[...] = (-jnp.log(-jnp.log(u1_ref[...]))
                           + jnp.log(-jnp.log(u0_ref[...])))

        Bk = B2_ref[pl.ds(k, _N), :]
        Dk = D2_ref[pl.ds(k, _N), :]
        ef = jnp.dot(jnp.maximum(A_ref[...] + Bk, 0.0).astype(bf),
                     W2ei_ref[...].astype(bf),
                     preferred_element_type=_F32) + b2ei_ref[...]
        mpre = (jnp.dot(ef.astype(bf), Wmef_ref[...].astype(bf),
                        preferred_element_type=_F32) + C_ref[...] + Dk)
        m = jnp.dot(jnp.maximum(mpre, 0.0).astype(bf), W2me_ref[...].astype(bf),
                    preferred_element_type=_F32) + b2me_ref[...]
        agg2[pl.ds(k, _N), :] = agg2[pl.ds(k, _N), :] + m

        ohrow = (lax.broadcasted_iota(jnp.int32, (1, _MEI), 1) == kidx).astype(_F32)
        mw = jnp.sum(m * wld_ref[...], axis=1, keepdims=True)
        ew_ref[...] = ew_ref[...] + mw * ohrow

        @pl.when(kidx == _MEI - 1)
        def _():
            d = (ew_ref[...] + bld_ref[0, 0] + gd_scr[...]) / tmp_ref[0, 0]
            ew_ref[...] = 1.0 / (1.0 + jnp.exp(-d))
            tail = jnp.concatenate(
                [agg2[pl.ds(_N, _MEI), :],
                 jnp.zeros((_N - _MEI, _LAT), _F32)], axis=0)
            aggd_ref[...] = agg2[pl.ds(0, _N), :] + tail

    full = lambda a: pl.BlockSpec(a.shape, lambda i: (0,) * a.ndim)
    args = (A, B2, C, D2, u0, u1,
            W2ei, b2ei, Wmef, W2me, b2me, wld, bld, temp)
    return pl.pallas_call(
        body,
        grid=(_MEI,),
        in_specs=[full(a) for a in args],
        out_specs=[
            pl.BlockSpec((_N, _MEI), lambda i: (0, 0)),
            pl.BlockSpec((_N, _LAT), lambda i: (0, 0)),
        ],
        out_shape=[
            jax.ShapeDtypeStruct((_N, _MEI), _F32),
            jax.ShapeDtypeStruct((_N, _LAT), _F32),
        ],
        scratch_shapes=[pltpu.VMEM((_N + _MEI, _LAT), _F32),
                        pltpu.VMEM((_N, _MEI), _F32)],
    )(*args)


def _tc_dense3(nfeat, aggd, W1mnf, W1mna, b1mn, W2mn, b2mn, Wrc, brc):
    def body(nf_ref, aggd_ref, W1mnf_ref, W1mna_ref, b1mn_ref,
             W2mn_ref, b2mn_ref, Wrc_ref, brc_ref, rc_ref):
        n2pre = (jnp.dot(nf_ref[...], W1mnf_ref[...], preferred_element_type=_F32)
                 + jnp.dot(aggd_ref[...], W1mna_ref[...], preferred_element_type=_F32)
                 + b1mn_ref[...])
        n2d = jnp.dot(jnp.maximum(n2pre, 0.0), W2mn_ref[...],
                      preferred_element_type=_F32) + b2mn_ref[...]
        rc_ref[...] = jnp.dot(n2d, Wrc_ref[...],
                              preferred_element_type=_F32) + brc_ref[...]

    return pl.pallas_call(
        body,
        out_shape=jax.ShapeDtypeStruct((_N, _DF), _F32),
    )(nfeat, aggd, W1mnf, W1mna, b1mn, W2mn, b2mn, Wrc, brc)


def kernel(nodes, edges, globals_, params, senders, receivers,
           n_node, n_edge, gumbel_temperature):
    nodes = nodes.astype(_F32)
    edges = edges.astype(_F32)
    globals_ = globals_.astype(_F32)
    s32 = senders.astype(jnp.int32)
    r32 = receivers.astype(jnp.int32)

    def edge_parts(p):
        W1, b1 = p['edge'][0]
        return (W1[:_DE], W1[_DE:_DE + _DF], W1[_DE + _DF:_DE + 2 * _DF],
                W1[_DE + 2 * _DF:], b1)
    We_e, Ws_e, Wr_e, Wg_e, b1_e = edge_parts(params['enc'])
    We_s, Ws_s, Wr_s, Wg_s, b1_s = edge_parts(params['sig'])
    We = jnp.concatenate([We_e, We_s], 1)
    Ws = jnp.concatenate([Ws_e, Ws_s], 1)
    Wr = jnp.concatenate([Wr_e, Wr_s], 1)
    Wg = jnp.concatenate([Wg_e, Wg_s], 1)
    b1 = jnp.concatenate([b1_e, b1_s]).reshape(1, -1)
    W2e_e, b2e_e = params['enc']['edge'][1]
    W2e_s, b2e_s = params['sig']['edge'][1]
    W2blk = jnp.zeros((2 * _DF, 2 * _LAT), _F32)
    W2blk = W2blk.at[:_DF, :_LAT].set(W2e_e).at[_DF:, _LAT:].set(W2e_s)
    b2 = jnp.concatenate([b2e_e, b2e_s]).reshape(1, -1)

    idx = jnp.concatenate([s32, r32], 0)
    gn = _sc_gather(idx, nodes)
    e2, e2sum = _tc_edge(gn, edges, globals_, Ws, Wr, We, Wg, b1, W2blk, b2)
    parts = _sc_scatter(e2, r32, jnp.zeros((_N, _DF), _F32))

    W1n_e, b1n_e = params['enc']['node'][0]
    W1n_s, b1n_s = params['sig']['node'][0]
    W2n_e, b2n_e = params['enc']['node'][1]
    W2n_s, b2n_s = params['sig']['node'][1]
    Wnn = jnp.concatenate([W1n_e[:_DF], W1n_s[:_DF]], 1)
    Wna = jnp.zeros((2 * _LAT, 2 * _DF), _F32)
    Wna = Wna.at[:_LAT, :_DF].set(W1n_e[_DF:_DF + _LAT])
    Wna = Wna.at[_LAT:, _DF:].set(W1n_s[_DF:_DF + _LAT])
    Wng = jnp.concatenate([W1n_e[_DF + _LAT:], W1n_s[_DF + _LAT:]], 1)
    b1n = jnp.concatenate([b1n_e, b1n_s]).reshape(1, -1)
    W2n = jnp.zeros((2 * _DF, 2 * _LAT), _F32)
    W2n = W2n.at[:_DF, :_LAT].set(W2n_e).at[_DF:, _LAT:].set(W2n_s)
    b2n = jnp.concatenate([b2n_e, b2n_s]).reshape(1, -1)

    G1e, b1ge = params['enc']['glob'][0]
    W2ge, b2ge = params['enc']['glob'][1]
    G1s, b1gs = params['sig']['glob'][0]
    W2gs, b2gs = params['sig']['glob'][1]

    dp = params['dec']
    W1d, b1d = dp['node_init'][0]
    W2d, b2d = dp['node_init'][1]
    Wz = W1d[:_LAT]
    wnn_row = W1d[_LAT].reshape(1, -1)
    wne_row = W1d[_LAT + 1].reshape(1, -1)
    wpos_row = W1d[_LAT + 2].reshape(1, -1)
    W1ei, b1ei = dp['edge_init'][0]
    W2ei, b2ei = dp['edge_init'][1]
    W1me, b1me = dp['mpg_edge'][0]
    W2me, b2me = dp['mpg_edge'][1]

    eps = jax.random.normal(jax.random.key(42), (1, _LAT), dtype=_F32)
    nnne = jnp.zeros((1, _DF), _F32)
    nnne = nnne.at[0, 0].set(n_node.astype(_F32)[0])
    nnne = nnne.at[0, 1].set(n_edge.astype(_F32)[0])

    mu, ls, nfeat, A, B, C, D = _tc_dense1(
        nodes, parts[0], parts[1], e2sum, globals_, eps, nnne,
        Wnn, Wna, Wng, b1n, W2n, b2n,
        G1e, b1ge.reshape(1, -1), W2ge, b2ge.reshape(1, -1),
        G1s, b1gs.reshape(1, -1), W2gs, b2gs.reshape(1, -1),
        Wz, wnn_row, wne_row, wpos_row, b1d.reshape(1, -1), W2d, b2d.reshape(1, -1),
        W1ei[:_DF], W1ei[_DF:], b1ei.reshape(1, -1),
        W1me[_DE:_DE + _DF], W1me[_DE + _DF:], b1me.reshape(1, -1))

    B2 = jnp.concatenate([B, B[:_MEI]], 0)
    D2 = jnp.concatenate([D, D[:_MEI]], 0)

    Wl, bl = dp['logit'][0]
    wld = (Wl[:, 1] - Wl[:, 0]).reshape(1, -1)
    bld = (bl[1] - bl[0]).reshape(1, 1)
    W1mn, b1mn = dp['mpg_node'][0]
    W2mn, b2mn = dp['mpg_node'][1]
    Wrc, brc = dp['recon'][0]

    u = jax.random.uniform(jax.random.key(43), (_N * _MEI, 2),
                           minval=1e-6, maxval=1.0 - 1e-6)
    u3 = u.reshape(_MEI, _N, 2)
    u0 = u3[:, :, 0].T
    u1 = u3[:, :, 1].T
    temp = jnp.asarray(gumbel_temperature, _F32).reshape(1, 1)

    ew, aggd = _tc_dense2(
        A, B2, C, D2, u0, u1,
        W2ei, b2ei.reshape(1, -1), W1me[:_DE], W2me, b2me.reshape(1, -1),
        wld, bld, temp)
    recon = _tc_dense3(
        nfeat, aggd, W1mn[:_DF], W1mn[_DF:], b1mn.reshape(1, -1),
        W2mn, b2mn.reshape(1, -1), Wrc, brc.reshape(1, -1))

    edge_weights = ew.T.reshape(-1)
    return (recon, edge_weights, mu, ls)


# speedup vs baseline: 5.6085x; 1.1254x over previous
import functools

import jax
import jax.numpy as jnp
from jax import lax
from jax.experimental import pallas as pl
from jax.experimental.pallas import tpu as pltpu
from jax.experimental.pallas import tpu_sc as plsc

_N = 10000
_E = 320000
_DF = 128
_DE = 16
_DG = 128
_LAT = 64
_MEI = 32

_F32 = jnp.float32

_NW = 32
_GPW = (2 * _E) // _NW
_GCH = 80
_GNCH = _GPW // _GCH


def _sc_gather(idx, table):
    mesh = plsc.VectorSubcoreMesh(core_axis_name="c", subcore_axis_name="s")

    @functools.partial(
        pl.kernel,
        out_type=jax.ShapeDtypeStruct((2 * _E, _DF), _F32),
        mesh=mesh,
        scratch_types=[
            pltpu.VMEM((_GPW,), jnp.int32),
            pltpu.VMEM((_GCH, _DF), _F32),
            pltpu.VMEM((_GCH, _DF), _F32),
            pltpu.SemaphoreType.DMA,
            pltpu.SemaphoreType.DMA,
            pltpu.SemaphoreType.DMA,
            pltpu.SemaphoreType.DMA,
        ],
    )
    def k(idx_hbm, tbl_hbm, out_hbm, idx_v, buf0, buf1, sg0, sg1, sw0, sw1):
        wid = lax.axis_index("s") * 2 + lax.axis_index("c")
        base = wid * _GPW
        pltpu.sync_copy(idx_hbm.at[pl.ds(base, _GPW)], idx_v)

        def start(j, buf, sem):
            pltpu.async_copy(tbl_hbm.at[idx_v.at[pl.ds(j * _GCH, _GCH)]], buf, sem)

        def wait(j, buf, sem):
            pltpu.make_async_copy(
                tbl_hbm.at[idx_v.at[pl.ds(j * _GCH, _GCH)]], buf, sem).wait()

        def wstart(j, buf, sem):
            pltpu.async_copy(buf, out_hbm.at[pl.ds(base + j * _GCH, _GCH)], sem)

        def wwait(j, buf, sem):
            pltpu.make_async_copy(
                buf, out_hbm.at[pl.ds(base + j * _GCH, _GCH)], sem).wait()

        start(0, buf0, sg0)
        start(1, buf1, sg1)

        def body(it, _):
            j0 = it * 2
            wait(j0, buf0, sg0)
            wstart(j0, buf0, sw0)
            wait(j0 + 1, buf1, sg1)
            wstart(j0 + 1, buf1, sw1)
            wwait(j0, buf0, sw0)

            @pl.when(it + 1 < _GNCH // 2)
            def _():
                start(j0 + 2, buf0, sg0)

            wwait(j0 + 1, buf1, sw1)

            @pl.when(it + 1 < _GNCH // 2)
            def _():
                start(j0 + 3, buf1, sg1)

            return 0

        lax.fori_loop(0, _GNCH // 2, body, 0)

    return k(idx, table)


_SPW = _E // _NW
_SCH = 80
_SNCH = _SPW // _SCH
_RPT = 624
_RTAIL = _N - 16 * _RPT


def _sc_scatter(e2, recv, zeros):
    mesh = plsc.VectorSubcoreMesh(core_axis_name="c", subcore_axis_name="s")

    @functools.partial(
        pl.kernel,
        out_type=jax.ShapeDtypeStruct((2, _N, _DF), _F32),
        mesh=mesh,
        scratch_types=[
            pltpu.VMEM_SHARED((_N, _DF), _F32),
            pltpu.VMEM((_SCH,), jnp.int32),
            pltpu.VMEM((_SCH,), jnp.int32),
            pltpu.VMEM((_SCH, _DF), _F32),
            pltpu.VMEM((_SCH, _DF), _F32),
            pltpu.SemaphoreType.DMA,
            pltpu.SemaphoreType.DMA,
            pltpu.SemaphoreType.DMA,
            pltpu.SemaphoreType.DMA,
        ],
    )
    def k(e2_hbm, recv_hbm, zeros_hbm, out_hbm, shared,
          idx0, idx1, rows0, rows1, si0, si1, sr0, sr1):
        cid = lax.axis_index("c")
        sid = lax.axis_index("s")
        pltpu.sync_copy(zeros_hbm.at[pl.ds(sid * _RPT, _RPT)],
                        shared.at[pl.ds(sid * _RPT, _RPT)])

        @pl.when(sid == 0)
        def _():
            pltpu.sync_copy(zeros_hbm.at[pl.ds(16 * _RPT, _RTAIL)],
                            shared.at[pl.ds(16 * _RPT, _RTAIL)])

        plsc.subcore_barrier()
        base = (sid * 2 + cid) * _SPW

        def lstart(j, idx_v, rows_v, si, sr):
            off = base + j * _SCH
            pltpu.async_copy(recv_hbm.at[pl.ds(off, _SCH)], idx_v, si)
            pltpu.async_copy(e2_hbm.at[pl.ds(off, _SCH)], rows_v, sr)

        def lwait(j, idx_v, rows_v, si, sr):
            off = base + j * _SCH
            pltpu.make_async_copy(recv_hbm.at[pl.ds(off, _SCH)], idx_v, si).wait()
            pltpu.make_async_copy(e2_hbm.at[pl.ds(off, _SCH)], rows_v, sr).wait()

        lstart(0, idx0, rows0, si0, sr0)
        lstart(1, idx1, rows1, si1, sr1)

        def body(it, _):
            j0 = it * 2
            lwait(j0, idx0, rows0, si0, sr0)
            pltpu.sync_copy(rows0, shared.at[idx0], add=True)

            @pl.when(it + 1 < _SNCH // 2)
            def _():
                lstart(j0 + 2, idx0, rows0, si0, sr0)

            lwait(j0 + 1, idx1, rows1, si1, sr1)
            pltpu.sync_copy(rows1, shared.at[idx1], add=True)

            @pl.when(it + 1 < _SNCH // 2)
            def _():
                lstart(j0 + 3, idx1, rows1, si1, sr1)

            return 0

        lax.fori_loop(0, _SNCH // 2, body, 0)
        if _SNCH % 2:
            jt = _SNCH - 1
            lstart(jt, idx0, rows0, si0, sr0)
            lwait(jt, idx0, rows0, si0, sr0)
            pltpu.sync_copy(rows0, shared.at[idx0], add=True)
        plsc.subcore_barrier()
        pltpu.sync_copy(shared.at[pl.ds(sid * _RPT, _RPT)],
                        out_hbm.at[cid].at[pl.ds(sid * _RPT, _RPT)])

        @pl.when(sid == 0)
        def _():
            pltpu.sync_copy(shared.at[pl.ds(16 * _RPT, _RTAIL)],
                            out_hbm.at[cid].at[pl.ds(16 * _RPT, _RTAIL)])

    return k(e2, recv, zeros)


_BE = 3200
_GSTEPS = _E // _BE


def _tc_edge(gn, edges, globals_, Ws, Wr, We, Wg, b1, W2blk, b2):
    def body(gs_ref, gr_ref, ed_ref, gl_ref, Ws_ref, Wr_ref, We_ref, Wg_ref,
             b1_ref, W2_ref, b2_ref, e2_ref, sum_ref):
        bf = jnp.bfloat16
        c = jnp.dot(gl_ref[...], Wg_ref[...], preferred_element_type=_F32) + b1_ref[...]
        pre = (jnp.dot(gs_ref[...].astype(bf), Ws_ref[...].astype(bf),
                       preferred_element_type=_F32)
               + jnp.dot(gr_ref[...].astype(bf), Wr_ref[...].astype(bf),
                         preferred_element_type=_F32)
               + jnp.dot(ed_ref[...].astype(bf), We_ref[...].astype(bf),
                         preferred_element_type=_F32)
               + c)
        e2 = jnp.dot(jnp.maximum(pre, 0.0).astype(bf), W2_ref[...].astype(bf),
                     preferred_element_type=_F32) + b2_ref[...]
        e2_ref[...] = e2
        s = jnp.sum(e2, axis=0, keepdims=True)

        @pl.when(pl.program_id(0) == 0)
        def _():
            sum_ref[...] = s

        @pl.when(pl.program_id(0) > 0)
        def _():
            sum_ref[...] = sum_ref[...] + s

    full = lambda a: pl.BlockSpec(a.shape, lambda i: (0,) * a.ndim)
    return pl.pallas_call(
        body,
        grid=(_GSTEPS,),
        in_specs=[
            pl.BlockSpec((_BE, _DF), lambda i: (i, 0)),
            pl.BlockSpec((_BE, _DF), lambda i: (i + _GSTEPS, 0)),
            pl.BlockSpec((_BE, _DE), lambda i: (i, 0)),
            full(globals_), full(Ws), full(Wr), full(We), full(Wg),
            full(b1), full(W2blk), full(b2),
        ],
        out_specs=[
            pl.BlockSpec((_BE, 2 * _LAT), lambda i: (i, 0)),
            pl.BlockSpec((1, 2 * _LAT), lambda i: (0, 0)),
        ],
        out_shape=[
            jax.ShapeDtypeStruct((_E, 2 * _LAT), _F32),
            jax.ShapeDtypeStruct((1, 2 * _LAT), _F32),
        ],
    )(gn, gn, edges, globals_, Ws, Wr, We, Wg, b1, W2blk, b2)


def _tc_dense1(nodes, ag0, ag1, e2sum, globals_, eps, nnne,
               Wnn, Wna, Wng, b1n, W2n, b2n,
               G1e, b1ge, W2ge, b2ge, G1s, b1gs, W2gs, b2gs,
               Wz, wnn_row, wne_row, wpos_row, b1d, W2d, b2d,
               W1ei_a, W1ei_b, b1ei, Wme_s, Wme_d, b1me):
    def body(nodes_ref, ag0_ref, ag1_ref, e2sum_ref, gl_ref, eps_ref, nnne_ref,
             Wnn_ref, Wna_ref, Wng_ref, b1n_ref, W2n_ref, b2n_ref,
             G1e_ref, b1ge_ref, W2ge_ref, b2ge_ref,
             G1s_ref, b1gs_ref, W2gs_ref, b2gs_ref,
             Wz_ref, wnn_ref, wne_ref, wpos_ref, b1d_ref, W2d_ref, b2d_ref,
             Wa_ref, Wb_ref, b1ei_ref, Wms_ref, Wmd_ref, b1me_ref,
             mu_ref, ls_ref, nf_ref, A_ref, B_ref, C_ref, D_ref):
        gl = gl_ref[...]
        agg = ag0_ref[...] + ag1_ref[...]
        npre = (jnp.dot(nodes_ref[...], Wnn_ref[...], preferred_element_type=_F32)
                + jnp.dot(agg, Wna_ref[...], preferred_element_type=_F32)
                + jnp.dot(gl, Wng_ref[...], preferred_element_type=_F32)
                + b1n_ref[...])
        n2 = jnp.dot(jnp.maximum(npre, 0.0), W2n_ref[...],
                     preferred_element_type=_F32) + b2n_ref[...]
        n2sum = jnp.sum(n2, axis=0, keepdims=True)
        e2sum = e2sum_ref[...]

        def glob(ns, es, G1, b1g, W2g, b2g):
            gin = jnp.concatenate([ns, es, gl], axis=1)
            gpre = jnp.dot(gin, G1, preferred_element_type=_F32) + b1g
            return jnp.dot(jnp.maximum(gpre, 0.0), W2g,
                           preferred_element_type=_F32) + b2g

        mu = glob(n2sum[:, :_LAT], e2sum[:, :_LAT],
                  G1e_ref[...], b1ge_ref[...], W2ge_ref[...], b2ge_ref[...])
        ls = glob(n2sum[:, _LAT:], e2sum[:, _LAT:],
                  G1s_ref[...], b1gs_ref[...], W2gs_ref[...], b2gs_ref[...])
        mu_ref[...] = mu
        ls_ref[...] = ls

        z = mu + jnp.exp(ls) * eps_ref[...]
        nn = nnne_ref[0, 0]
        ne = nnne_ref[0, 1]
        zc = (jnp.dot(z, Wz_ref[...], preferred_element_type=_F32)
              + nn * wnn_ref[...] + ne * wne_ref[...] + b1d_ref[...])
        pos = lax.broadcasted_iota(jnp.int32, (_N, 1), 0).astype(_F32) * (1.0 / _N)
        nf = jnp.dot(jnp.maximum(zc + pos * wpos_ref[...], 0.0), W2d_ref[...],
                     preferred_element_type=_F32) + b2d_ref[...]
        nf_ref[...] = nf
        A_ref[...] = jnp.dot(nf, Wa_ref[...], preferred_element_type=_F32) + b1ei_ref[...]
        B_ref[...] = jnp.dot(nf, Wb_ref[...], preferred_element_type=_F32)
        C_ref[...] = jnp.dot(nf, Wms_ref[...], preferred_element_type=_F32) + b1me_ref[...]
        D_ref[...] = jnp.dot(nf, Wmd_ref[...], preferred_element_type=_F32)

    return pl.pallas_call(
        body,
        out_shape=[
            jax.ShapeDtypeStruct((1, _LAT), _F32),
            jax.ShapeDtypeStruct((1, _LAT), _F32),
            jax.ShapeDtypeStruct((_N, _DF), _F32),
            jax.ShapeDtypeStruct((_N, _DF), _F32),
            jax.ShapeDtypeStruct((_N, _DF), _F32),
            jax.ShapeDtypeStruct((_N, _DF), _F32),
            jax.ShapeDtypeStruct((_N, _DF), _F32),
        ],
    )(nodes, ag0, ag1, e2sum, globals_, eps, nnne,
      Wnn, Wna, Wng, b1n, W2n, b2n,
      G1e, b1ge, W2ge, b2ge, G1s, b1gs, W2gs, b2gs,
      Wz, wnn_row, wne_row, wpos_row, b1d, W2d, b2d,
      W1ei_a, W1ei_b, b1ei, Wme_s, Wme_d, b1me)


def _tc_dense2(A, B2, C, D2, u0, u1,
               W2ei, b2ei, Wmef, W2me, b2me, wld, bld, temp):
    def body(A_ref, B2_ref, C_ref, D2_ref, u0_ref, u1_ref,
             W2ei_ref, b2ei_ref, Wmef_ref, W2me_ref, b2me_ref,
             wld_ref, bld_ref, tmp_ref,
             ew_ref, aggd_ref, agg2, gd_scr):
        kidx = pl.program_id(0)
        k = kidx + 1
        bf = jnp.bfloat16

        @pl.when(kidx == 0)
        def _():
            agg2[...] = jnp.zeros((_N + _MEI, _LAT), _F32)
            ew_ref[...] = jnp.zeros((_N, _MEI), _F32)
            gd_scr[...] = (-jnp.log(-jnp.log(u1_ref[...]))
                           + jnp.log(-jnp.log(u0_ref[...])))

        Bk = B2_ref[pl.ds(k, _N), :]
        Dk = D2_ref[pl.ds(k, _N), :]
        ef = jnp.dot(jnp.maximum(A_ref[...] + Bk, 0.0).astype(bf),
                     W2ei_ref[...].astype(bf),
                     preferred_element_type=_F32) + b2ei_ref[...]
        mpre = (jnp.dot(ef.astype(bf), Wmef_ref[...].astype(bf),
                        preferred_element_type=_F32) + C_ref[...] + Dk)
        m = jnp.dot(jnp.maximum(mpre, 0.0).astype(bf), W2me_ref[...].astype(bf),
                    preferred_element_type=_F32) + b2me_ref[...]
        agg2[pl.ds(k, _N), :] = agg2[pl.ds(k, _N), :] + m

        ohrow = (lax.broadcasted_iota(jnp.int32, (1, _MEI), 1) == kidx).astype(_F32)
        mw = jnp.sum(m * wld_ref[...], axis=1, keepdims=True)
        ew_ref[...] = ew_ref[...] + mw * ohrow

        @pl.when(kidx == _MEI - 1)
        def _():
            d = (ew_ref[...] + bld_ref[0, 0] + gd_scr[...]) / tmp_ref[0, 0]
            ew_ref[...] = 1.0 / (1.0 + jnp.exp(-d))
            tail = jnp.concatenate(
                [agg2[pl.ds(_N, _MEI), :],
                 jnp.zeros((_N - _MEI, _LAT), _F32)], axis=0)
            aggd_ref[...] = agg2[pl.ds(0, _N), :] + tail

    full = lambda a: pl.BlockSpec(a.shape, lambda i: (0,) * a.ndim)
    args = (A, B2, C, D2, u0, u1,
            W2ei, b2ei, Wmef, W2me, b2me, wld, bld, temp)
    return pl.pallas_call(
        body,
        grid=(_MEI,),
        in_specs=[full(a) for a in args],
        out_specs=[
            pl.BlockSpec((_N, _MEI), lambda i: (0, 0)),
            pl.BlockSpec((_N, _LAT), lambda i: (0, 0)),
        ],
        out_shape=[
            jax.ShapeDtypeStruct((_N, _MEI), _F32),
            jax.ShapeDtypeStruct((_N, _LAT), _F32),
        ],
        scratch_shapes=[pltpu.VMEM((_N + _MEI, _LAT), _F32),
                        pltpu.VMEM((_N, _MEI), _F32)],
    )(*args)


def _tc_dense3(nfeat, aggd, W1mnf, W1mna, b1mn, W2mn, b2mn, Wrc, brc):
    def body(nf_ref, aggd_ref, W1mnf_ref, W1mna_ref, b1mn_ref,
             W2mn_ref, b2mn_ref, Wrc_ref, brc_ref, rc_ref):
        n2pre = (jnp.dot(nf_ref[...], W1mnf_ref[...], preferred_element_type=_F32)
                 + jnp.dot(aggd_ref[...], W1mna_ref[...], preferred_element_type=_F32)
                 + b1mn_ref[...])
        n2d = jnp.dot(jnp.maximum(n2pre, 0.0), W2mn_ref[...],
                      preferred_element_type=_F32) + b2mn_ref[...]
        rc_ref[...] = jnp.dot(n2d, Wrc_ref[...],
                              preferred_element_type=_F32) + brc_ref[...]

    return pl.pallas_call(
        body,
        out_shape=jax.ShapeDtypeStruct((_N, _DF), _F32),
    )(nfeat, aggd, W1mnf, W1mna, b1mn, W2mn, b2mn, Wrc, brc)


def kernel(nodes, edges, globals_, params, senders, receivers,
           n_node, n_edge, gumbel_temperature):
    nodes = nodes.astype(_F32)
    edges = edges.astype(_F32)
    globals_ = globals_.astype(_F32)
    s32 = senders.astype(jnp.int32)
    r32 = receivers.astype(jnp.int32)

    def edge_parts(p):
        W1, b1 = p['edge'][0]
        return (W1[:_DE], W1[_DE:_DE + _DF], W1[_DE + _DF:_DE + 2 * _DF],
                W1[_DE + 2 * _DF:], b1)
    We_e, Ws_e, Wr_e, Wg_e, b1_e = edge_parts(params['enc'])
    We_s, Ws_s, Wr_s, Wg_s, b1_s = edge_parts(params['sig'])
    We = jnp.concatenate([We_e, We_s], 1)
    Ws = jnp.concatenate([Ws_e, Ws_s], 1)
    Wr = jnp.concatenate([Wr_e, Wr_s], 1)
    Wg = jnp.concatenate([Wg_e, Wg_s], 1)
    b1 = jnp.concatenate([b1_e, b1_s]).reshape(1, -1)
    W2e_e, b2e_e = params['enc']['edge'][1]
    W2e_s, b2e_s = params['sig']['edge'][1]
    W2blk = jnp.zeros((2 * _DF, 2 * _LAT), _F32)
    W2blk = W2blk.at[:_DF, :_LAT].set(W2e_e).at[_DF:, _LAT:].set(W2e_s)
    b2 = jnp.concatenate([b2e_e, b2e_s]).reshape(1, -1)

    idx = jnp.concatenate([s32, r32], 0)
    gn = _sc_gather(idx, nodes)
    e2, e2sum = _tc_edge(gn, edges, globals_, Ws, Wr, We, Wg, b1, W2blk, b2)
    parts = _sc_scatter(e2, r32, jnp.zeros((_N, _DF), _F32))

    W1n_e, b1n_e = params['enc']['node'][0]
    W1n_s, b1n_s = params['sig']['node'][0]
    W2n_e, b2n_e = params['enc']['node'][1]
    W2n_s, b2n_s = params['sig']['node'][1]
    Wnn = jnp.concatenate([W1n_e[:_DF], W1n_s[:_DF]], 1)
    Wna = jnp.zeros((2 * _LAT, 2 * _DF), _F32)
    Wna = Wna.at[:_LAT, :_DF].set(W1n_e[_DF:_DF + _LAT])
    Wna = Wna.at[_LAT:, _DF:].set(W1n_s[_DF:_DF + _LAT])
    Wng = jnp.concatenate([W1n_e[_DF + _LAT:], W1n_s[_DF + _LAT:]], 1)
    b1n = jnp.concatenate([b1n_e, b1n_s]).reshape(1, -1)
    W2n = jnp.zeros((2 * _DF, 2 * _LAT), _F32)
    W2n = W2n.at[:_DF, :_LAT].set(W2n_e).at[_DF:, _LAT:].set(W2n_s)
    b2n = jnp.concatenate([b2n_e, b2n_s]).reshape(1, -1)

    G1e, b1ge = params['enc']['glob'][0]
    W2ge, b2ge = params['enc']['glob'][1]
    G1s, b1gs = params['sig']['glob'][0]
    W2gs, b2gs = params['sig']['glob'][1]

    dp = params['dec']
    W1d, b1d = dp['node_init'][0]
    W2d, b2d = dp['node_init'][1]
    Wz = W1d[:_LAT]
    wnn_row = W1d[_LAT].reshape(1, -1)
    wne_row = W1d[_LAT + 1].reshape(1, -1)
    wpos_row = W1d[_LAT + 2].reshape(1, -1)
    W1ei, b1ei = dp['edge_init'][0]
    W2ei, b2ei = dp['edge_init'][1]
    W1me, b1me = dp['mpg_edge'][0]
    W2me, b2me = dp['mpg_edge'][1]

    eps = jax.random.normal(jax.random.key(42), (1, _LAT), dtype=_F32)
    nnne = jnp.zeros((1, _DF), _F32)
    nnne = nnne.at[0, 0].set(n_node.astype(_F32)[0])
    nnne = nnne.at[0, 1].set(n_edge.astype(_F32)[0])

    mu, ls, nfeat, A, B, C, D = _tc_dense1(
        nodes, parts[0], parts[1], e2sum, globals_, eps, nnne,
        Wnn, Wna, Wng, b1n, W2n, b2n,
        G1e, b1ge.reshape(1, -1), W2ge, b2ge.reshape(1, -1),
        G1s, b1gs.reshape(1, -1), W2gs, b2gs.reshape(1, -1),
        Wz, wnn_row, wne_row, wpos_row, b1d.reshape(1, -1), W2d, b2d.reshape(1, -1),
        W1ei[:_DF], W1ei[_DF:], b1ei.reshape(1, -1),
        W1me[_DE:_DE + _DF], W1me[_DE + _DF:], b1me.reshape(1, -1))

    B2 = jnp.concatenate([B, B[:_MEI]], 0)
    D2 = jnp.concatenate([D, D[:_MEI]], 0)

    Wl, bl = dp['logit'][0]
    wld = (Wl[:, 1] - Wl[:, 0]).reshape(1, -1)
    bld = (bl[1] - bl[0]).reshape(1, 1)
    W1mn, b1mn = dp['mpg_node'][0]
    W2mn, b2mn = dp['mpg_node'][1]
    Wrc, brc = dp['recon'][0]

    u = jax.random.uniform(jax.random.key(43), (_N * _MEI, 2),
                           minval=1e-6, maxval=1.0 - 1e-6)
    u3 = u.reshape(_MEI, _N, 2)
    u0 = u3[:, :, 0].T
    u1 = u3[:, :, 1].T
    temp = jnp.asarray(gumbel_temperature, _F32).reshape(1, 1)

    ew, aggd = _tc_dense2(
        A, B2, C, D2, u0, u1,
        W2ei, b2ei.reshape(1, -1), W1me[:_DE], W2me, b2me.reshape(1, -1),
        wld, bld, temp)
    recon = _tc_dense3(
        nfeat, aggd, W1mn[:_DF], W1mn[_DF:], b1mn.reshape(1, -1),
        W2mn, b2mn.reshape(1, -1), Wrc, brc.reshape(1, -1))

    edge_weights = ew.T.reshape(-1)
    return (recon, edge_weights, mu, ls)
